# Initial kernel scaffold; baseline (speedup 1.0000x reference)
import jax
import jax.numpy as jnp
from jax.experimental import pallas as pl


def kernel(pointcloud, params):
    raise NotImplementedError("write your pallas kernel here")


# speedup vs baseline: 5.3730x; 5.3730x over previous
import functools

import jax
import jax.numpy as jnp
import numpy as np
from jax.experimental import pallas as pl
from jax.experimental.pallas import tpu as pltpu

_NPOINTS = [1024, 256, 64]
_RADIUS = [[0.1, 0.2], [0.2, 0.4], [0.4, 0.8]]
_NSAMPLE = [[16, 32], [16, 32], [16, 32]]

_HIGH = jax.lax.Precision.HIGHEST


def _dot(a, b, dims, precision=_HIGH):
    return jax.lax.dot_general(a, b, (dims, ((), ())),
                               preferred_element_type=jnp.float32,
                               precision=precision)


def _fps_body(npoint, xyz_ref, new_t_ref):
    bsz, _, n = xyz_ref.shape
    x = xyz_ref[:, 0, :]
    y = xyz_ref[:, 1, :]
    z = xyz_ref[:, 2, :]
    iota_n = jax.lax.broadcasted_iota(jnp.int32, (bsz, n), 1)
    iota_p = jax.lax.broadcasted_iota(jnp.int32, (bsz, npoint), 1)

    def step(t, carry):
        dists, far, fx, fy, fz = carry
        oh = (iota_n == far).astype(jnp.float32)
        cx = jnp.sum(x * oh, axis=-1, keepdims=True)
        cy = jnp.sum(y * oh, axis=-1, keepdims=True)
        cz = jnp.sum(z * oh, axis=-1, keepdims=True)
        d = ((x - cx) ** 2 + (z - cz) ** 2) + (y - cy) ** 2
        dists = jnp.minimum(dists, d)
        mx = jnp.max(dists, axis=-1, keepdims=True)
        new_far = jnp.min(jnp.where(dists == mx, iota_n, n),
                          axis=-1, keepdims=True).astype(jnp.int32)
        sel = iota_p == t
        fx = jnp.where(sel, cx, fx)
        fy = jnp.where(sel, cy, fy)
        fz = jnp.where(sel, cz, fz)
        return dists, new_far, fx, fy, fz

    init = (jnp.full((bsz, n), 1e10, jnp.float32),
            jnp.zeros((bsz, 1), jnp.int32),
            jnp.zeros((bsz, npoint), jnp.float32),
            jnp.zeros((bsz, npoint), jnp.float32),
            jnp.zeros((bsz, npoint), jnp.float32))
    _, _, fx, fy, fz = jax.lax.fori_loop(0, npoint, step, init)
    new_t_ref[:, 0, :] = fx
    new_t_ref[:, 1, :] = fy
    new_t_ref[:, 2, :] = fz


def _fps(xyz_t, npoint):
    bsz = xyz_t.shape[0]
    return pl.pallas_call(
        functools.partial(_fps_body, npoint),
        out_shape=jax.ShapeDtypeStruct((bsz, 3, npoint), jnp.float32),
    )(xyz_t)


def _lane_cumsum(x):
    n = x.shape[-1]
    iota = jax.lax.broadcasted_iota(jnp.int32, x.shape, x.ndim - 1)
    s = 1
    while s < n:
        r = pltpu.roll(x, s, axis=x.ndim - 1)
        x = x + jnp.where(iota >= s, r, 0)
        s *= 2
    return x


def _group_body(r2, ns, pc, xyz_t_ref, new_ref, pts_ref, g_ref, s1_ref, s2_ref):
    n = xyz_t_ref.shape[2]
    c = pts_ref.shape[2]
    cx = new_ref[0]
    d2 = (cx[:, 0:1] - xyz_t_ref[0, 0:1, :]) ** 2
    d2 = d2 + (cx[:, 1:2] - xyz_t_ref[0, 1:2, :]) ** 2
    d2 = d2 + (cx[:, 2:3] - xyz_t_ref[0, 2:3, :]) ** 2
    mask = d2 <= r2
    rank = _lane_cumsum(mask.astype(jnp.int32))
    rank0 = rank - 1
    count = rank[:, n - 1:n]
    rows = []
    for k in range(ns):
        kk = jnp.where(count > k, k, 0)
        rows.append(jnp.logical_and(mask, rank0 == kk).astype(jnp.float32))
    oh = jnp.concatenate(rows, axis=0)
    g = _dot(oh, pts_ref[0], (((1,), (0,))))
    sub = jnp.concatenate([cx, jnp.zeros((pc, c - 3), jnp.float32)], axis=-1)
    g = g - jnp.concatenate([sub] * ns, axis=0)
    g_ref[...] = g
    gt = g.astype(jnp.bfloat16).astype(jnp.float32)
    p1 = jnp.sum(gt, axis=0, keepdims=True)
    p2 = _dot(gt, gt, (((0,), (0,))))
    first = jnp.logical_and(pl.program_id(0) == 0, pl.program_id(1) == 0)

    @pl.when(first)
    def _():
        s1_ref[...] = p1
        s2_ref[...] = p2

    @pl.when(jnp.logical_not(first))
    def _():
        s1_ref[...] += p1
        s2_ref[...] += p2


def _group(xyz_t, new_pts, pts, r2, ns, pc):
    bsz, _, n = xyz_t.shape
    npnt = new_pts.shape[1]
    c = pts.shape[2]
    nch = npnt // pc
    grid = (bsz, nch)
    return pl.pallas_call(
        functools.partial(_group_body, r2, ns, pc),
        grid=grid,
        in_specs=[
            pl.BlockSpec((1, 3, n), lambda b, ch: (b, 0, 0)),
            pl.BlockSpec((1, pc, 3), lambda b, ch: (b, ch, 0)),
            pl.BlockSpec((1, n, c), lambda b, ch: (b, 0, 0)),
        ],
        out_specs=[
            pl.BlockSpec((pc * ns, c), lambda b, ch: (b * nch + ch, 0)),
            pl.BlockSpec((1, c), lambda b, ch: (0, 0)),
            pl.BlockSpec((c, c), lambda b, ch: (0, 0)),
        ],
        out_shape=[
            jax.ShapeDtypeStruct((bsz * npnt * ns, c), jnp.float32),
            jax.ShapeDtypeStruct((1, c), jnp.float32),
            jax.ShapeDtypeStruct((c, c), jnp.float32),
        ],
    )(xyz_t, new_pts, pts)


def _affine_body(inv_n, wt_ref, ga_ref, be_ref, s1_ref, s2_ref, a_ref, b_ref):
    wt = wt_ref[...].astype(jnp.bfloat16).astype(jnp.float32)
    gm = s1_ref[...] * inv_n
    meany = _dot(gm, wt, (((1,), (0,))))
    u = _dot(s2_ref[...] * inv_n, wt, (((1,), (0,))))
    ey2 = jnp.sum(wt * u, axis=0, keepdims=True)
    var = ey2 - meany * meany
    a = ga_ref[...] / jnp.sqrt(var + 1e-5)
    a_ref[...] = a
    b_ref[...] = be_ref[...] - a * meany


def _affine(wt, gamma_r, beta_r, s1, s2, n_samples):
    o = wt.shape[1]
    return pl.pallas_call(
        functools.partial(_affine_body, 1.0 / n_samples),
        out_shape=[jax.ShapeDtypeStruct((1, o), jnp.float32),
                   jax.ShapeDtypeStruct((1, o), jnp.float32)],
    )(wt, gamma_r, beta_r, s1, s2)


def _layer_body(pool_ns, stats, x_ref, wt_ref, a_ref, b_ref, out_ref, *stat_refs):
    x = x_ref[...]
    y = _dot(x, wt_ref[...], (((1,), (0,))),
             precision=jax.lax.Precision.DEFAULT)
    xo = jnp.maximum(y * a_ref[...] + b_ref[...], 0.0)
    if pool_ns is None:
        out_ref[...] = xo
    else:
        ts, o = xo.shape
        out_ref[...] = jnp.max(xo.reshape(pool_ns, ts // pool_ns, o), axis=0)
    if stats:
        s1_ref, s2_ref = stat_refs
        xt = xo.astype(jnp.bfloat16).astype(jnp.float32)
        p1 = jnp.sum(xt, axis=0, keepdims=True)
        p2 = _dot(xt, xt, (((0,), (0,))))
        first = pl.program_id(0) == 0

        @pl.when(first)
        def _():
            s1_ref[...] = p1
            s2_ref[...] = p2

        @pl.when(jnp.logical_not(first))
        def _():
            s1_ref[...] += p1
            s2_ref[...] += p2


def _layer(x2, wt, a_r, b_r, ts, stats=False, pool_ns=None):
    s, c = x2.shape
    o = wt.shape[1]
    grid = (s // ts,)
    out_specs = []
    out_shape = []
    if pool_ns is None:
        out_specs.append(pl.BlockSpec((ts, o), lambda i: (i, 0)))
        out_shape.append(jax.ShapeDtypeStruct((s, o), jnp.float32))
    else:
        out_specs.append(pl.BlockSpec((ts // pool_ns, o), lambda i: (i, 0)))
        out_shape.append(jax.ShapeDtypeStruct((s // pool_ns, o), jnp.float32))
    if stats:
        out_specs.append(pl.BlockSpec((1, o), lambda i: (0, 0)))
        out_shape.append(jax.ShapeDtypeStruct((1, o), jnp.float32))
        out_specs.append(pl.BlockSpec((o, o), lambda i: (0, 0)))
        out_shape.append(jax.ShapeDtypeStruct((o, o), jnp.float32))
    res = pl.pallas_call(
        functools.partial(_layer_body, pool_ns, stats),
        grid=grid,
        in_specs=[
            pl.BlockSpec((ts, c), lambda i: (i, 0)),
            pl.BlockSpec((c, o), lambda i: (0, 0)),
            pl.BlockSpec((1, o), lambda i: (0, 0)),
            pl.BlockSpec((1, o), lambda i: (0, 0)),
        ],
        out_specs=out_specs,
        out_shape=out_shape,
    )(x2, wt, a_r, b_r)
    return res


def _run_mlp(x2, layers, s1, s2, pool_ns=None, pool_chunk=None,
             last_stats=False):
    n_samples = float(x2.shape[0])
    nl = len(layers)
    for li, lyr in enumerate(layers):
        wt = jnp.transpose(lyr["W"])
        a_r, b_r = _affine(wt, lyr["gamma"][None, :], lyr["beta"][None, :],
                           s1, s2, n_samples)
        last = li == nl - 1
        s = x2.shape[0]
        if last and pool_ns is not None:
            res = _layer(x2, wt, a_r, b_r, ts=pool_chunk, pool_ns=pool_ns)
            x2 = res[0]
        else:
            want = (not last) or last_stats
            res = _layer(x2, wt, a_r, b_r, ts=min(s, 4096), stats=want)
            x2 = res[0]
            if want:
                s1, s2 = res[1], res[2]
    return x2


def _interp_body(unk_ref, kt_ref, knf_ref, uf_ref, z_ref, s1_ref, s2_ref):
    nk = kt_ref.shape[2]
    u = unk_ref[0]
    d2 = (u[:, 0:1] - kt_ref[0, 0:1, :]) ** 2
    d2 = d2 + (u[:, 1:2] - kt_ref[0, 1:2, :]) ** 2
    d2 = d2 + (u[:, 2:3] - kt_ref[0, 2:3, :]) ** 2
    iota = jax.lax.broadcasted_iota(jnp.int32, d2.shape, 1)
    big = jnp.float32(3.0e38)
    dists = []
    idxs = []
    for _ in range(3):
        m = jnp.min(d2, axis=-1, keepdims=True)
        idx = jnp.min(jnp.where(d2 == m, iota, nk), axis=-1, keepdims=True)
        dists.append(m)
        idxs.append(idx)
        d2 = jnp.where(iota == idx, big, d2)
    dr = [1.0 / (m + 1e-8) for m in dists]
    norm = dr[0] + dr[1] + dr[2]
    wg = ((dr[0] / norm) * (iota == idxs[0]).astype(jnp.float32)
          + (dr[1] / norm) * (iota == idxs[1]).astype(jnp.float32)
          + (dr[2] / norm) * (iota == idxs[2]).astype(jnp.float32))
    interp = _dot(wg, knf_ref[0], (((1,), (0,))))
    z = jnp.concatenate([interp, uf_ref[0]], axis=-1)
    z_ref[...] = z
    zt = z.astype(jnp.bfloat16).astype(jnp.float32)
    p1 = jnp.sum(zt, axis=0, keepdims=True)
    p2 = _dot(zt, zt, (((0,), (0,))))
    first = jnp.logical_and(pl.program_id(0) == 0, pl.program_id(1) == 0)

    @pl.when(first)
    def _():
        s1_ref[...] = p1
        s2_ref[...] = p2

    @pl.when(jnp.logical_not(first))
    def _():
        s1_ref[...] += p1
        s2_ref[...] += p2


def _interp(unk_pts, known_t, knf, uf, pcu):
    bsz, nu, _ = unk_pts.shape
    nk = known_t.shape[2]
    ck = knf.shape[2]
    cu = uf.shape[2]
    ct = ck + cu
    nch = nu // pcu
    return pl.pallas_call(
        _interp_body,
        grid=(bsz, nch),
        in_specs=[
            pl.BlockSpec((1, pcu, 3), lambda b, ch: (b, ch, 0)),
            pl.BlockSpec((1, 3, nk), lambda b, ch: (b, 0, 0)),
            pl.BlockSpec((1, nk, ck), lambda b, ch: (b, 0, 0)),
            pl.BlockSpec((1, pcu, cu), lambda b, ch: (b, ch, 0)),
        ],
        out_specs=[
            pl.BlockSpec((pcu, ct), lambda b, ch: (b * nch + ch, 0)),
            pl.BlockSpec((1, ct), lambda b, ch: (0, 0)),
            pl.BlockSpec((ct, ct), lambda b, ch: (0, 0)),
        ],
        out_shape=[
            jax.ShapeDtypeStruct((bsz * nu, ct), jnp.float32),
            jax.ShapeDtypeStruct((1, ct), jnp.float32),
            jax.ShapeDtypeStruct((ct, ct), jnp.float32),
        ],
    )(unk_pts, known_t, knf, uf)


def kernel(pointcloud, params):
    bsz, n, _ = pointcloud.shape
    xyz_pts = pointcloud[..., 0:3]
    feat_pts = pointcloud[..., 3:]
    l_xyz_pts = [xyz_pts]
    l_xyz_t = [jnp.transpose(xyz_pts, (0, 2, 1))]
    l_feat_pts = [feat_pts]
    for i in range(len(_NPOINTS)):
        npnt = _NPOINTS[i]
        new_t = _fps(l_xyz_t[i], npnt)
        new_pts = jnp.transpose(new_t, (0, 2, 1))
        pts = jnp.concatenate([l_xyz_pts[i], l_feat_pts[i]], axis=-1)
        outs = []
        for j in range(len(_RADIUS[i])):
            ns = _NSAMPLE[i][j]
            r2 = np.float32(_RADIUS[i][j] * _RADIUS[i][j])
            pc = min(512 // ns, npnt)
            g2, s1, s2 = _group(l_xyz_t[i], new_pts, pts, r2, ns, pc)
            x2 = _run_mlp(g2, params["sa"][i][j], s1, s2, pool_ns=ns,
                          pool_chunk=pc * ns)
            outs.append(x2.reshape(bsz, npnt, -1))
        l_xyz_t.append(new_t)
        l_xyz_pts.append(new_pts)
        l_feat_pts.append(jnp.concatenate(outs, axis=-1))
    for i in range(-1, -(len(params["fp"]) + 1), -1):
        unk_pts = l_xyz_pts[i - 1]
        nu = unk_pts.shape[1]
        z2, s1, s2 = _interp(unk_pts, l_xyz_t[i], l_feat_pts[i],
                             l_feat_pts[i - 1], pcu=min(nu, 256))
        x2 = _run_mlp(z2, params["fp"][i], s1, s2)
        l_feat_pts[i - 1] = x2.reshape(bsz, nu, -1)
    return xyz_pts, jnp.transpose(l_feat_pts[0], (0, 2, 1))
